# speedup vs baseline: 27.6849x; 27.6849x over previous
import functools

import numpy as np
import jax
import jax.numpy as jnp
from jax.experimental import pallas as pl
from jax.experimental.pallas import tpu as pltpu

_T_VAL = 256.0
_S = float((np.sqrt(1.0 + 4.0 * _T_VAL ** 2) - 1.0) / (2.0 * _T_VAL ** 2))
_EPS = 1e-05
_W = 256


def _pcen_kernel(x_ref, At_ref, p_ref, a_ref, d_ref, rr_ref, drr_ref,
                 o_ref, carry_ref, *, precision):
    j = pl.program_id(1)
    x = x_ref[0]
    xt = x.T

    @pl.when(j == 0)
    def _():
        carry_ref[...] = xt[0:1, :]

    c = carry_ref[...]
    m = jnp.dot(At_ref[...], xt, preferred_element_type=jnp.float32,
                precision=precision)
    m = m + p_ref[...] * c
    carry_ref[...] = m[_W - 1:_W, :]

    smooth = jnp.exp(-a_ref[...] * jnp.log(m + _EPS))
    u = xt * smooth + d_ref[...]
    o_ref[0] = jnp.exp(rr_ref[...] * jnp.log(u)) - drr_ref[...]


def kernel(x, alpha, delta, r):
    B, C, F, T = x.shape
    BC = B * C
    NC = T // _W
    s = _S

    t_idx = np.arange(_W)
    dmat = t_idx[:, None] - t_idx[None, :]
    At = np.where(dmat >= 0, s * (1.0 - s) ** np.maximum(dmat, 0), 0.0)
    At = jnp.asarray(At, dtype=jnp.float32)
    p = jnp.asarray(((1.0 - s) ** (t_idx + 1.0)).reshape(_W, 1),
                    dtype=jnp.float32)

    a = jnp.exp(alpha).reshape(1, F)
    d = jnp.exp(delta).reshape(1, F)
    rr = jnp.exp(r).reshape(1, F)
    drr = jnp.exp(rr * delta).reshape(1, F)

    xr = x.reshape(BC, F, T)

    out = pl.pallas_call(
        functools.partial(_pcen_kernel, precision=jax.lax.Precision.HIGHEST),
        grid=(BC, NC),
        in_specs=[
            pl.BlockSpec((1, F, _W), lambda b, j: (b, 0, j)),
            pl.BlockSpec((_W, _W), lambda b, j: (0, 0)),
            pl.BlockSpec((_W, 1), lambda b, j: (0, 0)),
            pl.BlockSpec((1, F), lambda b, j: (0, 0)),
            pl.BlockSpec((1, F), lambda b, j: (0, 0)),
            pl.BlockSpec((1, F), lambda b, j: (0, 0)),
            pl.BlockSpec((1, F), lambda b, j: (0, 0)),
        ],
        out_specs=pl.BlockSpec((1, _W, F), lambda b, j: (b, j, 0)),
        out_shape=jax.ShapeDtypeStruct((BC, T, F), jnp.float32),
        scratch_shapes=[pltpu.VMEM((1, F), jnp.float32)],
        compiler_params=pltpu.CompilerParams(
            dimension_semantics=("parallel", "arbitrary"),
        ),
    )(xr, At, p, a, d, rr, drr)

    return out.reshape(B, C, T, F)


# speedup vs baseline: 31.2200x; 1.1277x over previous
import functools

import numpy as np
import jax
import jax.numpy as jnp
from jax.experimental import pallas as pl
from jax.experimental.pallas import tpu as pltpu

_T_VAL = 256.0
_S = float((np.sqrt(1.0 + 4.0 * _T_VAL ** 2) - 1.0) / (2.0 * _T_VAL ** 2))
_EPS = 1e-05
_W = 256


def _pcen_kernel(x_ref, At_ref, p_ref, a_ref, d_ref, rr_ref, drr_ref,
                 o_ref, carry_ref, *, precision):
    j = pl.program_id(1)
    x = x_ref[0]
    xt = x.T

    @pl.when(j == 0)
    def _():
        carry_ref[...] = xt[0:1, :]

    c = carry_ref[...]
    m = jnp.dot(At_ref[...], xt, preferred_element_type=jnp.float32,
                precision=precision)
    m = m + p_ref[...] * c
    carry_ref[...] = m[_W - 1:_W, :]

    smooth = jnp.exp(-a_ref[...] * jnp.log(m + _EPS))
    u = xt * smooth + d_ref[...]
    o_ref[0] = jnp.exp(rr_ref[...] * jnp.log(u)) - drr_ref[...]


def kernel(x, alpha, delta, r):
    B, C, F, T = x.shape
    BC = B * C
    NC = T // _W
    s = _S

    t_idx = np.arange(_W)
    dmat = t_idx[:, None] - t_idx[None, :]
    At = np.where(dmat >= 0, s * (1.0 - s) ** np.maximum(dmat, 0), 0.0)
    At = jnp.asarray(At, dtype=jnp.float32)
    p = jnp.asarray(((1.0 - s) ** (t_idx + 1.0)).reshape(_W, 1),
                    dtype=jnp.float32)

    a = jnp.exp(alpha).reshape(1, F)
    d = jnp.exp(delta).reshape(1, F)
    rr = jnp.exp(r).reshape(1, F)
    drr = jnp.exp(rr * delta).reshape(1, F)

    xr = x.reshape(BC, F, T)

    out = pl.pallas_call(
        functools.partial(_pcen_kernel, precision=jax.lax.Precision.DEFAULT),
        grid=(BC, NC),
        in_specs=[
            pl.BlockSpec((1, F, _W), lambda b, j: (b, 0, j)),
            pl.BlockSpec((_W, _W), lambda b, j: (0, 0)),
            pl.BlockSpec((_W, 1), lambda b, j: (0, 0)),
            pl.BlockSpec((1, F), lambda b, j: (0, 0)),
            pl.BlockSpec((1, F), lambda b, j: (0, 0)),
            pl.BlockSpec((1, F), lambda b, j: (0, 0)),
            pl.BlockSpec((1, F), lambda b, j: (0, 0)),
        ],
        out_specs=pl.BlockSpec((1, _W, F), lambda b, j: (b, j, 0)),
        out_shape=jax.ShapeDtypeStruct((BC, T, F), jnp.float32),
        scratch_shapes=[pltpu.VMEM((1, F), jnp.float32)],
        compiler_params=pltpu.CompilerParams(
            dimension_semantics=("parallel", "arbitrary"),
        ),
    )(xr, At, p, a, d, rr, drr)

    return out.reshape(B, C, T, F)


# speedup vs baseline: 151.0617x; 4.8386x over previous
import functools

import numpy as np
import jax
import jax.numpy as jnp
from jax.experimental import pallas as pl
from jax.experimental.pallas import tpu as pltpu

_T_VAL = 256.0
_S = float((np.sqrt(1.0 + 4.0 * _T_VAL ** 2) - 1.0) / (2.0 * _T_VAL ** 2))
_EPS = 1e-05
_W = 256


def _pcen_kernel(x_ref, At_ref, p_ref, a_ref, d_ref, rr_ref, drr_ref, o_ref):
    F, T = x_ref.shape[1], x_ref.shape[2]
    nck = T // _W
    At = At_ref[...]
    p = p_ref[...]
    a = a_ref[...]
    d = d_ref[...]
    rr = rr_ref[...]
    drr = drr_ref[...]

    c = None
    for j in range(nck):
        xj = x_ref[0, :, j * _W:(j + 1) * _W]
        xt = xj.T
        g = jnp.dot(At, xt, preferred_element_type=jnp.float32)
        if c is None:
            c = xt[0:1, :]
        m = g + p * c
        c = m[_W - 1:_W, :]
        smooth = jnp.exp(-a * jnp.log(m + _EPS))
        u = xt * smooth + d
        o_ref[0, j * _W:(j + 1) * _W, :] = jnp.exp(rr * jnp.log(u)) - drr


def kernel(x, alpha, delta, r):
    B, C, F, T = x.shape
    BC = B * C
    s = _S

    t_idx = np.arange(_W)
    dmat = t_idx[:, None] - t_idx[None, :]
    At = np.where(dmat >= 0, s * (1.0 - s) ** np.maximum(dmat, 0), 0.0)
    At = jnp.asarray(At, dtype=jnp.float32)
    p = jnp.asarray(((1.0 - s) ** (t_idx + 1.0)).reshape(_W, 1),
                    dtype=jnp.float32)

    a = jnp.exp(alpha).reshape(1, F)
    d = jnp.exp(delta).reshape(1, F)
    rr = jnp.exp(r).reshape(1, F)
    drr = jnp.exp(rr * delta).reshape(1, F)

    xr = x.reshape(BC, F, T)

    out = pl.pallas_call(
        _pcen_kernel,
        grid=(BC,),
        in_specs=[
            pl.BlockSpec((1, F, T), lambda b: (b, 0, 0)),
            pl.BlockSpec((_W, _W), lambda b: (0, 0)),
            pl.BlockSpec((_W, 1), lambda b: (0, 0)),
            pl.BlockSpec((1, F), lambda b: (0, 0)),
            pl.BlockSpec((1, F), lambda b: (0, 0)),
            pl.BlockSpec((1, F), lambda b: (0, 0)),
            pl.BlockSpec((1, F), lambda b: (0, 0)),
        ],
        out_specs=pl.BlockSpec((1, T, F), lambda b: (b, 0, 0)),
        out_shape=jax.ShapeDtypeStruct((BC, T, F), jnp.float32),
        compiler_params=pltpu.CompilerParams(
            dimension_semantics=("parallel",),
        ),
    )(xr, At, p, a, d, rr, drr)

    return out.reshape(B, C, T, F)
